# baseline (speedup 1.0000x reference)
import jax
import jax.numpy as jnp
from jax import lax
from jax.experimental import pallas as pl
from jax.experimental.pallas import tpu as pltpu
from jax.experimental.pallas import tpu_sc as plsc

N_NODES = 50000
N_EDGES = 800000
RBF_DIM = 128
DIM = 64
HDIM = DIM // 2
BETA = 0.5
THRESHOLD = 14.0


MLP_BLK = 2000


def _mlp_body(rbf_ref, w1_ref, b1_ref, w2_ref, b2_ref, h0_ref, h1_ref):
    x = rbf_ref[...]
    h = jnp.dot(x, w1_ref[...], preferred_element_type=jnp.float32) + b1_ref[...]
    bx = BETA * h
    sp = (jnp.maximum(bx, 0.0) + jnp.log1p(jnp.exp(-jnp.abs(bx)))) / BETA
    h = jnp.where(bx > THRESHOLD, h, sp)
    h = jnp.dot(h, w2_ref[...], preferred_element_type=jnp.float32) + b2_ref[...]
    h0_ref[...] = h[:, :HDIM]
    h1_ref[...] = h[:, HDIM:]


def _mlp(rbf, W1, b1, W2, b2):
    n = rbf.shape[0]
    grid = n // MLP_BLK
    return pl.pallas_call(
        _mlp_body,
        grid=(grid,),
        in_specs=[
            pl.BlockSpec((MLP_BLK, RBF_DIM), lambda i: (i, 0)),
            pl.BlockSpec((RBF_DIM, DIM), lambda i: (0, 0)),
            pl.BlockSpec((DIM,), lambda i: (0,)),
            pl.BlockSpec((DIM, DIM), lambda i: (0, 0)),
            pl.BlockSpec((DIM,), lambda i: (0,)),
        ],
        out_specs=[
            pl.BlockSpec((MLP_BLK, HDIM), lambda i: (i, 0)),
            pl.BlockSpec((MLP_BLK, HDIM), lambda i: (i, 0)),
        ],
        out_shape=[
            jax.ShapeDtypeStruct((n, HDIM), jnp.float32),
            jax.ShapeDtypeStruct((n, HDIM), jnp.float32),
        ],
    )(rbf, W1, b1, W2, b2)


NC = 2
NS = 16
CHUNK = 80
NBUF = 5
EDGES_PER_TILE = N_EDGES // NS
N_CHUNKS = EDGES_PER_TILE // CHUNK
N_GROUPS = N_CHUNKS // NBUF
HALF = N_NODES // NC
ACC_ROWS = 25088
ZROWS = ACC_ROWS // NS
OUT_ROWS = 1560


def _sc_body(nn0, nn1, h0, h1, ef0, ef1, src_hbm, dst_hbm, zero_hbm, out_hbm,
             src_v, dst_v, idx_v, nn_v, h_v, ef_v,
             sem_sd, sem_rows, sem_sc, acc_sh):
    c = lax.axis_index("c")
    s = lax.axis_index("s")
    base_node = c * HALF
    tile_e0 = s * EDGES_PER_TILE

    def issue_sd(g, b):
        e0 = tile_e0 + (jnp.minimum(g, N_GROUPS - 1) * NBUF + b) * CHUNK
        pltpu.async_copy(src_hbm.at[pl.ds(e0, CHUNK)], src_v[b], sem_sd[b])
        pltpu.async_copy(dst_hbm.at[pl.ds(e0, CHUNK)], dst_v[b], sem_sd[b])

    for half in range(2):
        nn_h, h_h, ef_h = ((nn0, h0, ef0), (nn1, h1, ef1))[half]

        pltpu.sync_copy(zero_hbm, acc_sh.at[pl.ds(s * ZROWS, ZROWS)])
        plsc.subcore_barrier()

        for b in range(NBUF):
            issue_sd(0, b)

        def group_body(g, _):
            for b in range(NBUF):
                @pl.when(g > 0)
                def _drain():
                    pltpu.make_async_copy(ef_v[b], acc_sh.at[idx_v[b]],
                                          sem_sc[b]).wait()
                pltpu.make_async_copy(src_hbm.at[pl.ds(0, CHUNK)], src_v[b],
                                      sem_sd[b]).wait()
                pltpu.make_async_copy(dst_hbm.at[pl.ds(0, CHUNK)], dst_v[b],
                                      sem_sd[b]).wait()
                e0 = tile_e0 + (g * NBUF + b) * CHUNK
                pltpu.async_copy(nn_h.at[src_v[b]], nn_v[b], sem_rows[b])
                pltpu.async_copy(h_h.at[pl.ds(e0, CHUNK)], h_v[b], sem_rows[b])
                pltpu.async_copy(ef_h.at[pl.ds(e0, CHUNK)], ef_v[b], sem_rows[b])
                for i in range(CHUNK // 16):
                    d = dst_v[b][pl.ds(i * 16, 16)]
                    ld = d - base_node
                    own = (ld >= 0) & (ld < HALF)
                    garb = HALF + jnp.bitwise_and(d, 63)
                    idx_v[b][pl.ds(i * 16, 16)] = jnp.where(own, ld, garb)

            for b in range(NBUF):
                for _ in range(3):
                    pltpu.make_async_copy(h_h.at[pl.ds(0, CHUNK)], h_v[b],
                                          sem_rows[b]).wait()

                def row_body(r, _):
                    for jc in range(HDIM // 16):
                        sl = pl.ds(jc * 16, 16)
                        ef_v[b][r, sl] = (nn_v[b][r, sl] * h_v[b][r, sl]
                                          + ef_v[b][r, sl])
                    return ()

                lax.fori_loop(0, CHUNK, row_body, (), unroll=8)
                pltpu.async_copy(ef_v[b], acc_sh.at[idx_v[b]], sem_sc[b],
                                 add=True)
                issue_sd(g + 1, b)
            return ()

        lax.fori_loop(0, N_GROUPS, group_body, ())
        for b in range(NBUF):
            pltpu.make_async_copy(ef_v[b], acc_sh.at[idx_v[b]], sem_sc[b]).wait()
            pltpu.make_async_copy(src_hbm.at[pl.ds(0, CHUNK)], src_v[b],
                                  sem_sd[b]).wait()
            pltpu.make_async_copy(dst_hbm.at[pl.ds(0, CHUNK)], dst_v[b],
                                  sem_sd[b]).wait()
        plsc.subcore_barrier()

        col = half * HDIM
        pltpu.sync_copy(acc_sh.at[pl.ds(s * OUT_ROWS, OUT_ROWS)],
                        out_hbm.at[pl.ds(base_node + s * OUT_ROWS, OUT_ROWS),
                                   pl.ds(col, HDIM)])

        @pl.when(s == 0)
        def _tail():
            pltpu.sync_copy(
                acc_sh.at[pl.ds(NS * OUT_ROWS, HALF - NS * OUT_ROWS)],
                out_hbm.at[pl.ds(base_node + NS * OUT_ROWS,
                                 HALF - NS * OUT_ROWS), pl.ds(col, HDIM)])

        if half == 0:
            plsc.subcore_barrier()


def _sc_scatter(nn0, nn1, h0, h1, ef0, ef1, src, dst, zeros):
    mesh = plsc.VectorSubcoreMesh(core_axis_name="c", subcore_axis_name="s")
    kfn = pl.kernel(
        _sc_body,
        out_type=jax.ShapeDtypeStruct((N_NODES, DIM), jnp.float32),
        mesh=mesh,
        compiler_params=pltpu.CompilerParams(use_tc_tiling_on_sc=False),
        scratch_types=[
            [pltpu.VMEM((CHUNK,), jnp.int32) for _ in range(NBUF)],
            [pltpu.VMEM((CHUNK,), jnp.int32) for _ in range(NBUF)],
            [pltpu.VMEM((CHUNK,), jnp.int32) for _ in range(NBUF)],
            [pltpu.VMEM((CHUNK, HDIM), jnp.float32) for _ in range(NBUF)],
            [pltpu.VMEM((CHUNK, HDIM), jnp.float32) for _ in range(NBUF)],
            [pltpu.VMEM((CHUNK, HDIM), jnp.float32) for _ in range(NBUF)],
            [pltpu.SemaphoreType.DMA for _ in range(NBUF)],
            [pltpu.SemaphoreType.DMA for _ in range(NBUF)],
            [pltpu.SemaphoreType.DMA for _ in range(NBUF)],
            pltpu.VMEM_SHARED((ACC_ROWS, HDIM), jnp.float32),
        ],
    )
    return kfn(nn0, nn1, h0, h1, ef0, ef1, src, dst, zeros)


def kernel(new_node, rbf, edge_f, edge_index, W1, b1, W2, b2):
    src = edge_index[0].astype(jnp.int32)
    dst = edge_index[1].astype(jnp.int32)
    h0, h1 = _mlp(rbf, W1, b1, W2, b2)
    nn0 = new_node[:, :HDIM]
    nn1 = new_node[:, HDIM:]
    ef0 = edge_f[:, :HDIM]
    ef1 = edge_f[:, HDIM:]
    zeros = jnp.zeros((ZROWS, HDIM), jnp.float32)
    return _sc_scatter(nn0, nn1, h0, h1, ef0, ef1, src, dst, zeros)


# speedup vs baseline: 1.3165x; 1.3165x over previous
import jax
import jax.numpy as jnp
from jax import lax
from jax.experimental import pallas as pl
from jax.experimental.pallas import tpu as pltpu
from jax.experimental.pallas import tpu_sc as plsc

N_NODES = 50000
N_EDGES = 800000
RBF_DIM = 128
DIM = 64
HDIM = DIM // 2
BETA = 0.5
THRESHOLD = 14.0


MLP_BLK = 2000


def _mlp_body(rbf_ref, w1_ref, b1_ref, w2_ref, b2_ref, h_ref):
    x = rbf_ref[...]
    h = jnp.dot(x, w1_ref[...], preferred_element_type=jnp.float32) + b1_ref[...]
    bx = BETA * h
    sp = (jnp.maximum(bx, 0.0) + jnp.log1p(jnp.exp(-jnp.abs(bx)))) / BETA
    h = jnp.where(bx > THRESHOLD, h, sp)
    h = jnp.dot(h, w2_ref[...], preferred_element_type=jnp.float32) + b2_ref[...]
    h_ref[...] = h


def _mlp(rbf, W1, b1, W2, b2):
    n = rbf.shape[0]
    grid = n // MLP_BLK
    return pl.pallas_call(
        _mlp_body,
        grid=(grid,),
        in_specs=[
            pl.BlockSpec((MLP_BLK, RBF_DIM), lambda i: (i, 0)),
            pl.BlockSpec((RBF_DIM, DIM), lambda i: (0, 0)),
            pl.BlockSpec((DIM,), lambda i: (0,)),
            pl.BlockSpec((DIM, DIM), lambda i: (0, 0)),
            pl.BlockSpec((DIM,), lambda i: (0,)),
        ],
        out_specs=pl.BlockSpec((MLP_BLK, DIM), lambda i: (i, 0)),
        out_shape=jax.ShapeDtypeStruct((n, DIM), jnp.float32),
    )(rbf, W1, b1, W2, b2)


NC = 2
NS = 16
CHUNK = 80
NBUF = 5
EDGES_PER_TILE = N_EDGES // NS
N_CHUNKS = EDGES_PER_TILE // CHUNK
N_GROUPS = N_CHUNKS // NBUF
HALF = N_NODES // NC
ACC_ROWS = 25088
ZROWS = ACC_ROWS // NS
OUT_ROWS = 1560


def _sc_body(nn0_hbm, nn1_hbm, h_hbm, ef_hbm, src_hbm, dst_hbm, zero_hbm,
             out_hbm,
             src_v, dst_v, idx_v, nn_v, h_v, ef_v,
             sem_sd, sem_rows, sem_sc, acc_sh):
    c = lax.axis_index("c")
    s = lax.axis_index("s")
    base_node = c * HALF
    tile_e0 = s * EDGES_PER_TILE

    def issue_sd(g, b):
        e0 = tile_e0 + (jnp.minimum(g, N_GROUPS - 1) * NBUF + b) * CHUNK
        pltpu.async_copy(src_hbm.at[pl.ds(e0, CHUNK)], src_v[b], sem_sd[b])
        pltpu.async_copy(dst_hbm.at[pl.ds(e0, CHUNK)], dst_v[b], sem_sd[b])

    for half in range(2):
        col = half * HDIM
        nn_h = (nn0_hbm, nn1_hbm)[half]

        pltpu.sync_copy(zero_hbm, acc_sh.at[pl.ds(s * ZROWS, ZROWS)])
        plsc.subcore_barrier()

        for b in range(NBUF):
            issue_sd(0, b)

        def group_body(g, _):
            for b in range(NBUF):
                @pl.when(g > 0)
                def _drain():
                    pltpu.make_async_copy(ef_v[b], acc_sh.at[idx_v[b]],
                                          sem_sc[b]).wait()
                pltpu.make_async_copy(src_hbm.at[pl.ds(0, CHUNK)], src_v[b],
                                      sem_sd[b]).wait()
                pltpu.make_async_copy(dst_hbm.at[pl.ds(0, CHUNK)], dst_v[b],
                                      sem_sd[b]).wait()
                e0 = tile_e0 + (g * NBUF + b) * CHUNK
                pltpu.async_copy(nn_h.at[src_v[b]], nn_v[b], sem_rows[b])
                pltpu.async_copy(h_hbm.at[pl.ds(e0, CHUNK), pl.ds(col, HDIM)],
                                 h_v[b], sem_rows[b])
                pltpu.async_copy(ef_hbm.at[pl.ds(e0, CHUNK), pl.ds(col, HDIM)],
                                 ef_v[b], sem_rows[b])
                for i in range(CHUNK // 16):
                    d = dst_v[b][pl.ds(i * 16, 16)]
                    ld = d - base_node
                    own = (ld >= 0) & (ld < HALF)
                    garb = HALF + jnp.bitwise_and(d, 63)
                    idx_v[b][pl.ds(i * 16, 16)] = jnp.where(own, ld, garb)

            for b in range(NBUF):
                for _ in range(3):
                    pltpu.make_async_copy(h_hbm.at[pl.ds(0, CHUNK),
                                                   pl.ds(col, HDIM)],
                                          h_v[b], sem_rows[b]).wait()

                def row_body(r, _):
                    for jc in range(HDIM // 16):
                        sl = pl.ds(jc * 16, 16)
                        ef_v[b][r, sl] = (nn_v[b][r, sl] * h_v[b][r, sl]
                                          + ef_v[b][r, sl])
                    return ()

                lax.fori_loop(0, CHUNK, row_body, (), unroll=8)
                pltpu.async_copy(ef_v[b], acc_sh.at[idx_v[b]], sem_sc[b],
                                 add=True)
                issue_sd(g + 1, b)
            return ()

        lax.fori_loop(0, N_GROUPS, group_body, ())
        for b in range(NBUF):
            pltpu.make_async_copy(ef_v[b], acc_sh.at[idx_v[b]], sem_sc[b]).wait()
            pltpu.make_async_copy(src_hbm.at[pl.ds(0, CHUNK)], src_v[b],
                                  sem_sd[b]).wait()
            pltpu.make_async_copy(dst_hbm.at[pl.ds(0, CHUNK)], dst_v[b],
                                  sem_sd[b]).wait()
        plsc.subcore_barrier()

        pltpu.sync_copy(acc_sh.at[pl.ds(s * OUT_ROWS, OUT_ROWS)],
                        out_hbm.at[pl.ds(base_node + s * OUT_ROWS, OUT_ROWS),
                                   pl.ds(col, HDIM)])

        @pl.when(s == 0)
        def _tail():
            pltpu.sync_copy(
                acc_sh.at[pl.ds(NS * OUT_ROWS, HALF - NS * OUT_ROWS)],
                out_hbm.at[pl.ds(base_node + NS * OUT_ROWS,
                                 HALF - NS * OUT_ROWS), pl.ds(col, HDIM)])

        if half == 0:
            plsc.subcore_barrier()


def _sc_scatter(nn0, nn1, h, edge_f, src, dst, zeros):
    mesh = plsc.VectorSubcoreMesh(core_axis_name="c", subcore_axis_name="s")
    kfn = pl.kernel(
        _sc_body,
        out_type=jax.ShapeDtypeStruct((N_NODES, DIM), jnp.float32),
        mesh=mesh,
        compiler_params=pltpu.CompilerParams(use_tc_tiling_on_sc=False),
        scratch_types=[
            [pltpu.VMEM((CHUNK,), jnp.int32) for _ in range(NBUF)],
            [pltpu.VMEM((CHUNK,), jnp.int32) for _ in range(NBUF)],
            [pltpu.VMEM((CHUNK,), jnp.int32) for _ in range(NBUF)],
            [pltpu.VMEM((CHUNK, HDIM), jnp.float32) for _ in range(NBUF)],
            [pltpu.VMEM((CHUNK, HDIM), jnp.float32) for _ in range(NBUF)],
            [pltpu.VMEM((CHUNK, HDIM), jnp.float32) for _ in range(NBUF)],
            [pltpu.SemaphoreType.DMA for _ in range(NBUF)],
            [pltpu.SemaphoreType.DMA for _ in range(NBUF)],
            [pltpu.SemaphoreType.DMA for _ in range(NBUF)],
            pltpu.VMEM_SHARED((ACC_ROWS, HDIM), jnp.float32),
        ],
    )
    return kfn(nn0, nn1, h, edge_f, src, dst, zeros)


def kernel(new_node, rbf, edge_f, edge_index, W1, b1, W2, b2):
    src = edge_index[0].astype(jnp.int32)
    dst = edge_index[1].astype(jnp.int32)
    h = _mlp(rbf, W1, b1, W2, b2)
    nn0 = new_node[:, :HDIM]
    nn1 = new_node[:, HDIM:]
    zeros = jnp.zeros((ZROWS, HDIM), jnp.float32)
    return _sc_scatter(nn0, nn1, h, edge_f, src, dst, zeros)


# speedup vs baseline: 1.3688x; 1.0398x over previous
import jax
import jax.numpy as jnp
from jax import lax
from jax.experimental import pallas as pl
from jax.experimental.pallas import tpu as pltpu
from jax.experimental.pallas import tpu_sc as plsc

N_NODES = 50000
N_EDGES = 800000
RBF_DIM = 128
DIM = 64
HDIM = DIM // 2
BETA = 0.5
THRESHOLD = 14.0


MLP_BLK = 4000


def _mlp_body(rbf_ref, w1_ref, b1_ref, w2_ref, b2_ref, h_ref):
    x = rbf_ref[...].astype(jnp.bfloat16)
    h = jnp.dot(x, w1_ref[...].astype(jnp.bfloat16),
                preferred_element_type=jnp.float32) + b1_ref[...]
    bx = BETA * h
    sp = (jnp.maximum(bx, 0.0) + jnp.log1p(jnp.exp(-jnp.abs(bx)))) / BETA
    h = jnp.where(bx > THRESHOLD, h, sp)
    h = jnp.dot(h.astype(jnp.bfloat16), w2_ref[...].astype(jnp.bfloat16),
                preferred_element_type=jnp.float32) + b2_ref[...]
    h_ref[...] = h


def _mlp(rbf, W1, b1, W2, b2):
    n = rbf.shape[0]
    grid = n // MLP_BLK
    return pl.pallas_call(
        _mlp_body,
        grid=(grid,),
        in_specs=[
            pl.BlockSpec((MLP_BLK, RBF_DIM), lambda i: (i, 0)),
            pl.BlockSpec((RBF_DIM, DIM), lambda i: (0, 0)),
            pl.BlockSpec((DIM,), lambda i: (0,)),
            pl.BlockSpec((DIM, DIM), lambda i: (0, 0)),
            pl.BlockSpec((DIM,), lambda i: (0,)),
        ],
        out_specs=pl.BlockSpec((MLP_BLK, DIM), lambda i: (i, 0)),
        out_shape=jax.ShapeDtypeStruct((n, DIM), jnp.float32),
    )(rbf, W1, b1, W2, b2)


NC = 2
NS = 16
CHUNK = 80
NBUF = 5
EDGES_PER_TILE = N_EDGES // NS
N_CHUNKS = EDGES_PER_TILE // CHUNK
N_GROUPS = N_CHUNKS // NBUF
HALF = N_NODES // NC
ACC_ROWS = 25088
ZROWS = ACC_ROWS // NS
OUT_ROWS = 1560


def _sc_body(nn0_hbm, nn1_hbm, h_hbm, ef_hbm, src_hbm, dst_hbm, zero_hbm,
             out_hbm,
             src_v, dst_v, idx_v, nn_v, h_v, ef_v,
             sem_sd, sem_rows, sem_sc, acc_sh):
    c = lax.axis_index("c")
    s = lax.axis_index("s")
    base_node = c * HALF
    tile_e0 = s * EDGES_PER_TILE

    def issue_sd(g, b):
        e0 = tile_e0 + (jnp.minimum(g, N_GROUPS - 1) * NBUF + b) * CHUNK
        pltpu.async_copy(src_hbm.at[pl.ds(e0, CHUNK)], src_v[b], sem_sd[b])
        pltpu.async_copy(dst_hbm.at[pl.ds(e0, CHUNK)], dst_v[b], sem_sd[b])

    for half in range(2):
        col = half * HDIM
        nn_h = (nn0_hbm, nn1_hbm)[half]

        pltpu.sync_copy(zero_hbm, acc_sh.at[pl.ds(s * ZROWS, ZROWS)])
        plsc.subcore_barrier()

        for b in range(NBUF):
            issue_sd(0, b)

        def group_body(g, _):
            for b in range(NBUF):
                @pl.when(g > 0)
                def _drain():
                    pltpu.make_async_copy(ef_v[b], acc_sh.at[idx_v[b]],
                                          sem_sc[b]).wait()
                pltpu.make_async_copy(src_hbm.at[pl.ds(0, CHUNK)], src_v[b],
                                      sem_sd[b]).wait()
                pltpu.make_async_copy(dst_hbm.at[pl.ds(0, CHUNK)], dst_v[b],
                                      sem_sd[b]).wait()
                e0 = tile_e0 + (g * NBUF + b) * CHUNK
                pltpu.async_copy(nn_h.at[src_v[b]], nn_v[b], sem_rows[b])
                pltpu.async_copy(h_hbm.at[pl.ds(e0, CHUNK), pl.ds(col, HDIM)],
                                 h_v[b], sem_rows[b])
                pltpu.async_copy(ef_hbm.at[pl.ds(e0, CHUNK), pl.ds(col, HDIM)],
                                 ef_v[b], sem_rows[b])
                for i in range(CHUNK // 16):
                    d = dst_v[b][pl.ds(i * 16, 16)]
                    ld = d - base_node
                    own = (ld >= 0) & (ld < HALF)
                    garb = HALF + jnp.bitwise_and(d, 63)
                    idx_v[b][pl.ds(i * 16, 16)] = jnp.where(own, ld, garb)

            for b in range(NBUF):
                for _ in range(3):
                    pltpu.make_async_copy(h_hbm.at[pl.ds(0, CHUNK),
                                                   pl.ds(col, HDIM)],
                                          h_v[b], sem_rows[b]).wait()

                def row_body(r, _):
                    for jc in range(HDIM // 16):
                        sl = pl.ds(jc * 16, 16)
                        ef_v[b][r, sl] = (nn_v[b][r, sl] * h_v[b][r, sl]
                                          + ef_v[b][r, sl])
                    return ()

                lax.fori_loop(0, CHUNK, row_body, (), unroll=8)
                pltpu.async_copy(ef_v[b], acc_sh.at[idx_v[b]], sem_sc[b],
                                 add=True)
                issue_sd(g + 1, b)
            return ()

        lax.fori_loop(0, N_GROUPS, group_body, ())
        for b in range(NBUF):
            pltpu.make_async_copy(ef_v[b], acc_sh.at[idx_v[b]], sem_sc[b]).wait()
            pltpu.make_async_copy(src_hbm.at[pl.ds(0, CHUNK)], src_v[b],
                                  sem_sd[b]).wait()
            pltpu.make_async_copy(dst_hbm.at[pl.ds(0, CHUNK)], dst_v[b],
                                  sem_sd[b]).wait()
        plsc.subcore_barrier()

        pltpu.sync_copy(acc_sh.at[pl.ds(s * OUT_ROWS, OUT_ROWS)],
                        out_hbm.at[pl.ds(base_node + s * OUT_ROWS, OUT_ROWS),
                                   pl.ds(col, HDIM)])

        @pl.when(s == 0)
        def _tail():
            pltpu.sync_copy(
                acc_sh.at[pl.ds(NS * OUT_ROWS, HALF - NS * OUT_ROWS)],
                out_hbm.at[pl.ds(base_node + NS * OUT_ROWS,
                                 HALF - NS * OUT_ROWS), pl.ds(col, HDIM)])

        if half == 0:
            plsc.subcore_barrier()


def _sc_scatter(nn0, nn1, h, edge_f, src, dst, zeros):
    mesh = plsc.VectorSubcoreMesh(core_axis_name="c", subcore_axis_name="s")
    kfn = pl.kernel(
        _sc_body,
        out_type=jax.ShapeDtypeStruct((N_NODES, DIM), jnp.float32),
        mesh=mesh,
        compiler_params=pltpu.CompilerParams(use_tc_tiling_on_sc=False),
        scratch_types=[
            [pltpu.VMEM((CHUNK,), jnp.int32) for _ in range(NBUF)],
            [pltpu.VMEM((CHUNK,), jnp.int32) for _ in range(NBUF)],
            [pltpu.VMEM((CHUNK,), jnp.int32) for _ in range(NBUF)],
            [pltpu.VMEM((CHUNK, HDIM), jnp.float32) for _ in range(NBUF)],
            [pltpu.VMEM((CHUNK, HDIM), jnp.float32) for _ in range(NBUF)],
            [pltpu.VMEM((CHUNK, HDIM), jnp.float32) for _ in range(NBUF)],
            [pltpu.SemaphoreType.DMA for _ in range(NBUF)],
            [pltpu.SemaphoreType.DMA for _ in range(NBUF)],
            [pltpu.SemaphoreType.DMA for _ in range(NBUF)],
            pltpu.VMEM_SHARED((ACC_ROWS, HDIM), jnp.float32),
        ],
    )
    return kfn(nn0, nn1, h, edge_f, src, dst, zeros)


def kernel(new_node, rbf, edge_f, edge_index, W1, b1, W2, b2):
    src = edge_index[0].astype(jnp.int32)
    dst = edge_index[1].astype(jnp.int32)
    h = _mlp(rbf, W1, b1, W2, b2)
    nn0 = new_node[:, :HDIM]
    nn1 = new_node[:, HDIM:]
    zeros = jnp.zeros((ZROWS, HDIM), jnp.float32)
    return _sc_scatter(nn0, nn1, h, edge_f, src, dst, zeros)


# speedup vs baseline: 1.4443x; 1.0551x over previous
import jax
import jax.numpy as jnp
from jax import lax
from jax.experimental import pallas as pl
from jax.experimental.pallas import tpu as pltpu
from jax.experimental.pallas import tpu_sc as plsc

N_NODES = 50000
N_EDGES = 800000
RBF_DIM = 128
DIM = 64
HDIM = DIM // 2
BETA = 0.5
THRESHOLD = 14.0


MLP_BLK = 4000


def _mlp_body(rbf_ref, w1_ref, b1_ref, w2_ref, b2_ref, h_ref):
    x = rbf_ref[...].astype(jnp.bfloat16)
    h = jnp.dot(x, w1_ref[...].astype(jnp.bfloat16),
                preferred_element_type=jnp.float32) + b1_ref[...]
    bx = BETA * h
    sp = (jnp.maximum(bx, 0.0) + jnp.log1p(jnp.exp(-jnp.abs(bx)))) / BETA
    h = jnp.where(bx > THRESHOLD, h, sp)
    h = jnp.dot(h.astype(jnp.bfloat16), w2_ref[...].astype(jnp.bfloat16),
                preferred_element_type=jnp.float32) + b2_ref[...]
    h_ref[...] = h


def _mlp(rbf, W1, b1, W2, b2):
    n = rbf.shape[0]
    grid = n // MLP_BLK
    return pl.pallas_call(
        _mlp_body,
        grid=(grid,),
        in_specs=[
            pl.BlockSpec((MLP_BLK, RBF_DIM), lambda i: (i, 0)),
            pl.BlockSpec((RBF_DIM, DIM), lambda i: (0, 0)),
            pl.BlockSpec((DIM,), lambda i: (0,)),
            pl.BlockSpec((DIM, DIM), lambda i: (0, 0)),
            pl.BlockSpec((DIM,), lambda i: (0,)),
        ],
        out_specs=pl.BlockSpec((MLP_BLK, DIM), lambda i: (i, 0)),
        out_shape=jax.ShapeDtypeStruct((n, DIM), jnp.float32),
    )(rbf, W1, b1, W2, b2)


NC = 2
NS = 16
CHUNK = 80
NBUF = 5
SPLIT = 384000
HALF = N_NODES // NC
ACC_ROWS = 25088
ZROWS = ACC_ROWS // NS
PAD_ROWS = NC * ACC_ROWS
OUT_ROWS = 1560


def _make_sc_body(ept, edge_base, chained):
    n_groups = ept // CHUNK // NBUF

    def body(nn0_hbm, nn1_hbm, h_hbm, ef_hbm, src_hbm, dst_hbm, init_hbm,
             out_hbm, src_v, dst_v, idx_v, nn_v, h_v, ef_v,
             sem_sd, sem_rows, sem_sc, acc_sh):
        c = lax.axis_index("c")
        s = lax.axis_index("s")
        base_node = c * HALF
        tile_e0 = s * ept

        def issue_sd(g, b):
            e0 = (edge_base + tile_e0
                  + (jnp.minimum(g, n_groups - 1) * NBUF + b) * CHUNK)
            pltpu.async_copy(src_hbm.at[pl.ds(e0, CHUNK)], src_v[b], sem_sd[b])
            pltpu.async_copy(dst_hbm.at[pl.ds(e0, CHUNK)], dst_v[b], sem_sd[b])

        for half in range(2):
            col = half * HDIM
            nn_h = (nn0_hbm, nn1_hbm)[half]

            if chained:
                pltpu.sync_copy(
                    init_hbm.at[pl.ds(c * ACC_ROWS + s * ZROWS, ZROWS),
                                pl.ds(col, HDIM)],
                    acc_sh.at[pl.ds(s * ZROWS, ZROWS)])
            else:
                pltpu.sync_copy(init_hbm, acc_sh.at[pl.ds(s * ZROWS, ZROWS)])
            plsc.subcore_barrier()

            for b in range(NBUF):
                issue_sd(0, b)

            def group_body(g, _):
                for b in range(NBUF):
                    @pl.when(g > 0)
                    def _drain():
                        pltpu.make_async_copy(ef_v[b], acc_sh.at[idx_v[b]],
                                              sem_sc[b]).wait()
                    pltpu.make_async_copy(src_hbm.at[pl.ds(0, CHUNK)], src_v[b],
                                          sem_sd[b]).wait()
                    pltpu.make_async_copy(dst_hbm.at[pl.ds(0, CHUNK)], dst_v[b],
                                          sem_sd[b]).wait()
                    e0 = tile_e0 + (g * NBUF + b) * CHUNK
                    pltpu.async_copy(nn_h.at[src_v[b]], nn_v[b], sem_rows[b])
                    pltpu.async_copy(h_hbm.at[pl.ds(e0, CHUNK),
                                              pl.ds(col, HDIM)],
                                     h_v[b], sem_rows[b])
                    pltpu.async_copy(ef_hbm.at[pl.ds(edge_base + e0, CHUNK),
                                               pl.ds(col, HDIM)],
                                     ef_v[b], sem_rows[b])
                    for i in range(CHUNK // 16):
                        d = dst_v[b][pl.ds(i * 16, 16)]
                        ld = d - base_node
                        own = (ld >= 0) & (ld < HALF)
                        garb = HALF + jnp.bitwise_and(d, 63)
                        idx_v[b][pl.ds(i * 16, 16)] = jnp.where(own, ld, garb)

                for b in range(NBUF):
                    for _ in range(3):
                        pltpu.make_async_copy(h_hbm.at[pl.ds(0, CHUNK),
                                                       pl.ds(col, HDIM)],
                                              h_v[b], sem_rows[b]).wait()

                    def row_body(r, _):
                        for jc in range(HDIM // 16):
                            sl = pl.ds(jc * 16, 16)
                            ef_v[b][r, sl] = (nn_v[b][r, sl] * h_v[b][r, sl]
                                              + ef_v[b][r, sl])
                        return ()

                    lax.fori_loop(0, CHUNK, row_body, (), unroll=8)
                    pltpu.async_copy(ef_v[b], acc_sh.at[idx_v[b]], sem_sc[b],
                                     add=True)
                    issue_sd(g + 1, b)
                return ()

            lax.fori_loop(0, n_groups, group_body, ())
            for b in range(NBUF):
                pltpu.make_async_copy(ef_v[b], acc_sh.at[idx_v[b]],
                                      sem_sc[b]).wait()
                pltpu.make_async_copy(src_hbm.at[pl.ds(0, CHUNK)], src_v[b],
                                      sem_sd[b]).wait()
                pltpu.make_async_copy(dst_hbm.at[pl.ds(0, CHUNK)], dst_v[b],
                                      sem_sd[b]).wait()
            plsc.subcore_barrier()

            if chained:
                pltpu.sync_copy(
                    acc_sh.at[pl.ds(s * OUT_ROWS, OUT_ROWS)],
                    out_hbm.at[pl.ds(base_node + s * OUT_ROWS, OUT_ROWS),
                               pl.ds(col, HDIM)])

                @pl.when(s == 0)
                def _tail():
                    pltpu.sync_copy(
                        acc_sh.at[pl.ds(NS * OUT_ROWS, HALF - NS * OUT_ROWS)],
                        out_hbm.at[pl.ds(base_node + NS * OUT_ROWS,
                                         HALF - NS * OUT_ROWS),
                                   pl.ds(col, HDIM)])
            else:
                pltpu.sync_copy(
                    acc_sh.at[pl.ds(s * ZROWS, ZROWS)],
                    out_hbm.at[pl.ds(c * ACC_ROWS + s * ZROWS, ZROWS),
                               pl.ds(col, HDIM)])

            if half == 0:
                plsc.subcore_barrier()

    return body


def _sc_scatter(nn0, nn1, h_a, h_b, edge_f, src, dst, zeros):
    mesh = plsc.VectorSubcoreMesh(core_axis_name="c", subcore_axis_name="s")
    scratch = [
        [pltpu.VMEM((CHUNK,), jnp.int32) for _ in range(NBUF)],
        [pltpu.VMEM((CHUNK,), jnp.int32) for _ in range(NBUF)],
        [pltpu.VMEM((CHUNK,), jnp.int32) for _ in range(NBUF)],
        [pltpu.VMEM((CHUNK, HDIM), jnp.float32) for _ in range(NBUF)],
        [pltpu.VMEM((CHUNK, HDIM), jnp.float32) for _ in range(NBUF)],
        [pltpu.VMEM((CHUNK, HDIM), jnp.float32) for _ in range(NBUF)],
        [pltpu.SemaphoreType.DMA for _ in range(NBUF)],
        [pltpu.SemaphoreType.DMA for _ in range(NBUF)],
        [pltpu.SemaphoreType.DMA for _ in range(NBUF)],
        pltpu.VMEM_SHARED((ACC_ROWS, HDIM), jnp.float32),
    ]
    params = pltpu.CompilerParams(use_tc_tiling_on_sc=False)
    kfn_a = pl.kernel(
        _make_sc_body(SPLIT // NS, 0, chained=False),
        out_type=jax.ShapeDtypeStruct((PAD_ROWS, DIM), jnp.float32),
        mesh=mesh, compiler_params=params, scratch_types=scratch)
    pad = kfn_a(nn0, nn1, h_a, edge_f, src, dst, zeros)
    kfn_b = pl.kernel(
        _make_sc_body((N_EDGES - SPLIT) // NS, SPLIT, chained=True),
        out_type=jax.ShapeDtypeStruct((N_NODES, DIM), jnp.float32),
        mesh=mesh, compiler_params=params, scratch_types=scratch)
    return kfn_b(nn0, nn1, h_b, edge_f, src, dst, pad)


def kernel(new_node, rbf, edge_f, edge_index, W1, b1, W2, b2):
    src = edge_index[0].astype(jnp.int32)
    dst = edge_index[1].astype(jnp.int32)
    h_a = _mlp(rbf[:SPLIT], W1, b1, W2, b2)
    h_b = _mlp(rbf[SPLIT:], W1, b1, W2, b2)
    nn0 = new_node[:, :HDIM]
    nn1 = new_node[:, HDIM:]
    zeros = jnp.zeros((ZROWS, HDIM), jnp.float32)
    return _sc_scatter(nn0, nn1, h_a, h_b, edge_f, src, dst, zeros)


# speedup vs baseline: 1.7319x; 1.1991x over previous
import jax
import jax.numpy as jnp
from jax import lax
from jax.experimental import pallas as pl
from jax.experimental.pallas import tpu as pltpu
from jax.experimental.pallas import tpu_sc as plsc

N_NODES = 50000
N_EDGES = 800000
RBF_DIM = 128
DIM = 64
HDIM = DIM // 2
BETA = 0.5
THRESHOLD = 14.0


MLP_BLK = 4000


def _mlp_body(rbf_ref, w1_ref, b1_ref, w2_ref, b2_ref, h_ref):
    x = rbf_ref[...].astype(jnp.bfloat16)
    h = jnp.dot(x, w1_ref[...].astype(jnp.bfloat16),
                preferred_element_type=jnp.float32) + b1_ref[...]
    bx = BETA * h
    sp = (jnp.maximum(bx, 0.0) + jnp.log1p(jnp.exp(-jnp.abs(bx)))) / BETA
    h = jnp.where(bx > THRESHOLD, h, sp)
    h = jnp.dot(h.astype(jnp.bfloat16), w2_ref[...].astype(jnp.bfloat16),
                preferred_element_type=jnp.float32) + b2_ref[...]
    h_ref[...] = h


def _mlp(rbf, W1, b1, W2, b2, base, n):
    grid = n // MLP_BLK
    base_blk = base // MLP_BLK
    return pl.pallas_call(
        _mlp_body,
        grid=(grid,),
        in_specs=[
            pl.BlockSpec((MLP_BLK, RBF_DIM), lambda i: (i + base_blk, 0)),
            pl.BlockSpec((RBF_DIM, DIM), lambda i: (0, 0)),
            pl.BlockSpec((DIM,), lambda i: (0,)),
            pl.BlockSpec((DIM, DIM), lambda i: (0, 0)),
            pl.BlockSpec((DIM,), lambda i: (0,)),
        ],
        out_specs=pl.BlockSpec((MLP_BLK, DIM), lambda i: (i, 0)),
        out_shape=jax.ShapeDtypeStruct((n, DIM), jnp.float32),
    )(rbf, W1, b1, W2, b2)


NC = 2
NS = 16
CHUNK = 80
NBUF = 5
SPLIT = 384000
HALF = N_NODES // NC
ACC_ROWS = 25088
ZROWS = ACC_ROWS // NS
PAD_ROWS = NC * ACC_ROWS
OUT_ROWS = 1560


def _ef_body(ef_hbm, dst_hbm, zero_hbm, out_hbm,
             dst_v, idx_v, ef_v, sem_sd, sem_rows, sem_sc, acc_sh):
    c = lax.axis_index("c")
    s = lax.axis_index("s")
    base_node = c * HALF
    ept = N_EDGES // NS
    nb = 2
    n_groups = 312

    def comp_idx(b):
        for i in range(CHUNK // 16):
            d = dst_v[b][pl.ds(i * 16, 16)]
            ld = d - base_node
            own = (ld >= 0) & (ld < HALF)
            garb = HALF + jnp.bitwise_and(d, 63)
            idx_v[b][pl.ds(i * 16, 16)] = jnp.where(own, ld, garb)

    def issue_sd(g, b):
        e0 = s * ept + (jnp.minimum(g, n_groups - 1) * nb + b) * CHUNK
        pltpu.async_copy(dst_hbm.at[pl.ds(e0, CHUNK)], dst_v[b], sem_sd[b])

    pltpu.sync_copy(zero_hbm, acc_sh.at[pl.ds(s * ZROWS, ZROWS)])
    plsc.subcore_barrier()
    for b in range(nb):
        issue_sd(0, b)

    def group_body(g, _):
        for b in range(nb):
            @pl.when(g > 0)
            def _drain():
                pltpu.make_async_copy(ef_v[b], acc_sh.at[idx_v[b]],
                                      sem_sc[b]).wait()
            pltpu.make_async_copy(dst_hbm.at[pl.ds(0, CHUNK)], dst_v[b],
                                  sem_sd[b]).wait()
            e0 = s * ept + (g * nb + b) * CHUNK
            pltpu.async_copy(ef_hbm.at[pl.ds(e0, CHUNK)], ef_v[b], sem_rows[b])
            comp_idx(b)
        for b in range(nb):
            pltpu.make_async_copy(ef_hbm.at[pl.ds(0, CHUNK)], ef_v[b],
                                  sem_rows[b]).wait()
            pltpu.async_copy(ef_v[b], acc_sh.at[idx_v[b]], sem_sc[b], add=True)
            issue_sd(g + 1, b)
        return ()

    lax.fori_loop(0, n_groups, group_body, ())
    for b in range(nb):
        pltpu.make_async_copy(ef_v[b], acc_sh.at[idx_v[b]], sem_sc[b]).wait()
        pltpu.make_async_copy(dst_hbm.at[pl.ds(0, CHUNK)], dst_v[b],
                              sem_sd[b]).wait()
    e0t = s * ept + 624 * CHUNK
    pltpu.sync_copy(dst_hbm.at[pl.ds(e0t, CHUNK)], dst_v[0])
    comp_idx(0)
    pltpu.sync_copy(ef_hbm.at[pl.ds(e0t, CHUNK)], ef_v[0])
    pltpu.sync_copy(ef_v[0], acc_sh.at[idx_v[0]], add=True)
    plsc.subcore_barrier()
    pltpu.sync_copy(acc_sh.at[pl.ds(s * ZROWS, ZROWS)],
                    out_hbm.at[pl.ds(c * ACC_ROWS + s * ZROWS, ZROWS)])


def _make_sc_body(ept, edge_base, final):
    n_groups = ept // CHUNK // NBUF

    def body(nn0_hbm, nn1_hbm, h_hbm, src_hbm, dst_hbm, init_hbm,
             out_hbm, src_v, dst_v, idx_v, nn_v, h_v,
             sem_sd, sem_rows, sem_sc, acc_sh):
        c = lax.axis_index("c")
        s = lax.axis_index("s")
        base_node = c * HALF
        tile_e0 = s * ept

        def issue_sd(g, b):
            e0 = (edge_base + tile_e0
                  + (jnp.minimum(g, n_groups - 1) * NBUF + b) * CHUNK)
            pltpu.async_copy(src_hbm.at[pl.ds(e0, CHUNK)], src_v[b], sem_sd[b])
            pltpu.async_copy(dst_hbm.at[pl.ds(e0, CHUNK)], dst_v[b], sem_sd[b])

        for half in range(2):
            col = half * HDIM
            nn_h = (nn0_hbm, nn1_hbm)[half]

            pltpu.sync_copy(
                init_hbm.at[pl.ds(c * ACC_ROWS + s * ZROWS, ZROWS),
                            pl.ds(col, HDIM)],
                acc_sh.at[pl.ds(s * ZROWS, ZROWS)])
            plsc.subcore_barrier()

            for b in range(NBUF):
                issue_sd(0, b)

            def group_body(g, _):
                for b in range(NBUF):
                    @pl.when(g > 0)
                    def _drain():
                        pltpu.make_async_copy(h_v[b], acc_sh.at[idx_v[b]],
                                              sem_sc[b]).wait()
                    pltpu.make_async_copy(src_hbm.at[pl.ds(0, CHUNK)], src_v[b],
                                          sem_sd[b]).wait()
                    pltpu.make_async_copy(dst_hbm.at[pl.ds(0, CHUNK)], dst_v[b],
                                          sem_sd[b]).wait()
                    e0 = tile_e0 + (g * NBUF + b) * CHUNK
                    pltpu.async_copy(nn_h.at[src_v[b]], nn_v[b], sem_rows[b])
                    pltpu.async_copy(h_hbm.at[pl.ds(e0, CHUNK),
                                              pl.ds(col, HDIM)],
                                     h_v[b], sem_rows[b])
                    for i in range(CHUNK // 16):
                        d = dst_v[b][pl.ds(i * 16, 16)]
                        ld = d - base_node
                        own = (ld >= 0) & (ld < HALF)
                        garb = HALF + jnp.bitwise_and(d, 63)
                        idx_v[b][pl.ds(i * 16, 16)] = jnp.where(own, ld, garb)

                for b in range(NBUF):
                    for _ in range(2):
                        pltpu.make_async_copy(h_hbm.at[pl.ds(0, CHUNK),
                                                       pl.ds(col, HDIM)],
                                              h_v[b], sem_rows[b]).wait()

                    def row_body(r, _):
                        for jc in range(HDIM // 16):
                            sl = pl.ds(jc * 16, 16)
                            h_v[b][r, sl] = nn_v[b][r, sl] * h_v[b][r, sl]
                        return ()

                    lax.fori_loop(0, CHUNK, row_body, (), unroll=8)
                    pltpu.async_copy(h_v[b], acc_sh.at[idx_v[b]], sem_sc[b],
                                     add=True)
                    issue_sd(g + 1, b)
                return ()

            lax.fori_loop(0, n_groups, group_body, ())
            for b in range(NBUF):
                pltpu.make_async_copy(h_v[b], acc_sh.at[idx_v[b]],
                                      sem_sc[b]).wait()
                pltpu.make_async_copy(src_hbm.at[pl.ds(0, CHUNK)], src_v[b],
                                      sem_sd[b]).wait()
                pltpu.make_async_copy(dst_hbm.at[pl.ds(0, CHUNK)], dst_v[b],
                                      sem_sd[b]).wait()
            plsc.subcore_barrier()

            if final:
                pltpu.sync_copy(
                    acc_sh.at[pl.ds(s * OUT_ROWS, OUT_ROWS)],
                    out_hbm.at[pl.ds(base_node + s * OUT_ROWS, OUT_ROWS),
                               pl.ds(col, HDIM)])

                @pl.when(s == 0)
                def _tail():
                    pltpu.sync_copy(
                        acc_sh.at[pl.ds(NS * OUT_ROWS, HALF - NS * OUT_ROWS)],
                        out_hbm.at[pl.ds(base_node + NS * OUT_ROWS,
                                         HALF - NS * OUT_ROWS),
                                   pl.ds(col, HDIM)])
            else:
                pltpu.sync_copy(
                    acc_sh.at[pl.ds(s * ZROWS, ZROWS)],
                    out_hbm.at[pl.ds(c * ACC_ROWS + s * ZROWS, ZROWS),
                               pl.ds(col, HDIM)])

            if half == 0:
                plsc.subcore_barrier()

    return body


def _sc_scatter(nn0, nn1, h_a, h_b, edge_f, src, dst, zeros64):
    mesh = plsc.VectorSubcoreMesh(core_axis_name="c", subcore_axis_name="s")
    scratch_ef = [
        [pltpu.VMEM((CHUNK,), jnp.int32) for _ in range(2)],
        [pltpu.VMEM((CHUNK,), jnp.int32) for _ in range(2)],
        [pltpu.VMEM((CHUNK, DIM), jnp.float32) for _ in range(2)],
        [pltpu.SemaphoreType.DMA for _ in range(2)],
        [pltpu.SemaphoreType.DMA for _ in range(2)],
        [pltpu.SemaphoreType.DMA for _ in range(2)],
        pltpu.VMEM_SHARED((ACC_ROWS, DIM), jnp.float32),
    ]
    kfn_ef = pl.kernel(
        _ef_body,
        out_type=jax.ShapeDtypeStruct((PAD_ROWS, DIM), jnp.float32),
        mesh=mesh,
        compiler_params=pltpu.CompilerParams(use_tc_tiling_on_sc=False),
        scratch_types=scratch_ef)
    pad0 = kfn_ef(edge_f, dst, zeros64)

    def mk_scratch():
        return [
            [pltpu.VMEM((CHUNK,), jnp.int32) for _ in range(NBUF)],
            [pltpu.VMEM((CHUNK,), jnp.int32) for _ in range(NBUF)],
            [pltpu.VMEM((CHUNK,), jnp.int32) for _ in range(NBUF)],
            [pltpu.VMEM((CHUNK, HDIM), jnp.float32) for _ in range(NBUF)],
            [pltpu.VMEM((CHUNK, HDIM), jnp.float32) for _ in range(NBUF)],
            [pltpu.SemaphoreType.DMA for _ in range(NBUF)],
            [pltpu.SemaphoreType.DMA for _ in range(NBUF)],
            [pltpu.SemaphoreType.DMA for _ in range(NBUF)],
            pltpu.VMEM_SHARED((ACC_ROWS, HDIM), jnp.float32),
        ]

    kfn_a = pl.kernel(
        _make_sc_body(SPLIT // NS, 0, final=False),
        out_type=jax.ShapeDtypeStruct((PAD_ROWS, DIM), jnp.float32),
        mesh=mesh,
        compiler_params=pltpu.CompilerParams(use_tc_tiling_on_sc=False),
        scratch_types=mk_scratch())
    pad = kfn_a(nn0, nn1, h_a, src, dst, pad0)
    kfn_b = pl.kernel(
        _make_sc_body((N_EDGES - SPLIT) // NS, SPLIT, final=True),
        out_type=jax.ShapeDtypeStruct((N_NODES, DIM), jnp.float32),
        mesh=mesh,
        compiler_params=pltpu.CompilerParams(use_tc_tiling_on_sc=False),
        scratch_types=mk_scratch())
    return kfn_b(nn0, nn1, h_b, src, dst, pad)


def kernel(new_node, rbf, edge_f, edge_index, W1, b1, W2, b2):
    src = edge_index[0].astype(jnp.int32)
    dst = edge_index[1].astype(jnp.int32)
    h_a = _mlp(rbf, W1, b1, W2, b2, 0, SPLIT)
    h_b = _mlp(rbf, W1, b1, W2, b2, SPLIT, N_EDGES - SPLIT)
    nn0 = new_node[:, :HDIM]
    nn1 = new_node[:, HDIM:]
    zeros64 = jnp.zeros((ZROWS, DIM), jnp.float32)
    return _sc_scatter(nn0, nn1, h_a, h_b, edge_f, src, dst, zeros64)
